# Initial kernel scaffold; baseline (speedup 1.0000x reference)
import jax
import jax.numpy as jnp
from jax.experimental import pallas as pl


def kernel(z_h, edge_index_h_h, pos_world, eW1, eb1, eW2, eb2, wW1, wb1, wW2, wb2, nW1, nb1, nW2, nb2, gW1, gb1, gW2, gb2):
    raise NotImplementedError("write your pallas kernel here")


# speedup vs baseline: 2.7885x; 2.7885x over previous
import functools

import jax
import jax.numpy as jnp
from jax import lax
from jax.experimental import pallas as pl
from jax.experimental.pallas import tpu as pltpu, tpu_sc as plsc

N = 10000
E = 320000
F = 128

NW = 32
EPW = E // NW
GB = 80
NGB = EPW // GB
RPS = 1000

def _gather_body(z_hbm, src_hbm, tgt_hbm, out_s, out_t, sidx, tidx, bufs,
                 buft, sem_s, sem_t):
    wid = lax.axis_index("s") * 2 + lax.axis_index("c")
    base = wid * EPW
    pltpu.sync_copy(src_hbm.at[wid], sidx)
    pltpu.sync_copy(tgt_hbm.at[wid], tidx)

    def body(j, carry):
        cs = pltpu.async_copy(z_hbm.at[sidx.at[j]], bufs, sem_s)
        ct = pltpu.async_copy(z_hbm.at[tidx.at[j]], buft, sem_t)
        cs.wait()
        pltpu.sync_copy(bufs, out_s.at[pl.ds(base + j * GB, GB)])
        ct.wait()
        pltpu.sync_copy(buft, out_t.at[pl.ds(base + j * GB, GB)])
        return carry

    lax.fori_loop(0, NGB, body, 0)


def _scatter_body(val_hbm, tgt_hbm, zero_hbm, out_hbm, tidx, vbuf, acc):
    c = lax.axis_index("c")
    s = lax.axis_index("s")
    wid = s * 2 + c
    base = wid * EPW

    @pl.when(s < N // RPS)
    def _():
        pltpu.sync_copy(zero_hbm.at[pl.ds(s * RPS, RPS)],
                        acc.at[pl.ds(s * RPS, RPS)])

    plsc.subcore_barrier()
    pltpu.sync_copy(tgt_hbm.at[wid], tidx)

    def body(j, carry):
        pltpu.sync_copy(val_hbm.at[pl.ds(base + j * GB, GB)], vbuf)
        pltpu.sync_copy(vbuf, acc.at[tidx.at[j]], add=True)
        return carry

    lax.fori_loop(0, NGB, body, 0)
    plsc.subcore_barrier()

    @pl.when(s < N // RPS)
    def _():
        pltpu.sync_copy(acc.at[pl.ds(s * RPS, RPS)],
                        out_hbm.at[c, pl.ds(s * RPS, RPS)])


@functools.cache
def _sc_kernels():
    mesh = plsc.VectorSubcoreMesh(core_axis_name="c", subcore_axis_name="s")
    gather = pl.kernel(
        _gather_body,
        out_type=(
            jax.ShapeDtypeStruct((E, F), jnp.float32),
            jax.ShapeDtypeStruct((E, F), jnp.float32),
        ),
        mesh=mesh,
        scratch_types=[
            pltpu.VMEM((NGB, GB), jnp.int32),
            pltpu.VMEM((NGB, GB), jnp.int32),
            pltpu.VMEM((GB, F), jnp.float32),
            pltpu.VMEM((GB, F), jnp.float32),
            pltpu.SemaphoreType.DMA,
            pltpu.SemaphoreType.DMA,
        ],
    )
    scatter = pl.kernel(
        _scatter_body,
        out_type=jax.ShapeDtypeStruct((2, N, F), jnp.float32),
        mesh=mesh,
        scratch_types=[
            pltpu.VMEM((NGB, GB), jnp.int32),
            pltpu.VMEM((GB, F), jnp.float32),
            pltpu.VMEM_SHARED((N, F), jnp.float32),
        ],
    )
    return gather, scatter


BE = 2000


def _edge_body(gs_ref, gt_ref, w1a_ref, w1b_ref, w1r_ref, b1_ref,
               ew2_ref, eb2_ref, ww2_ref, wb2_ref, out_ref):
    zs = gs_ref[...]
    zt = gt_ref[...]
    d0 = zs[:, 0:1] - zt[:, 0:1]
    d1 = zs[:, 1:2] - zt[:, 1:2]
    d2 = zs[:, 2:3] - zt[:, 2:3]
    dist = d0 * d0 + d1 * d1 + d2 * d2
    a0, a1, a2 = zs[:, 3:4], zs[:, 4:5], zs[:, 5:6]
    b0, b1v, b2 = zt[:, 3:4], zt[:, 4:5], zt[:, 5:6]
    c0 = a1 * b2 - a2 * b1v
    c1 = a2 * b0 - a0 * b2
    c2 = a0 * b1v - a1 * b0
    cn = jnp.sqrt(c0 * c0 + c1 * c1 + c2 * c2)
    h = jnp.dot(zs, w1a_ref[...], preferred_element_type=jnp.float32)
    h = h + jnp.dot(zt, w1b_ref[...], preferred_element_type=jnp.float32)
    feats = (d0, d1, d2, dist, c0, c1, c2, cn)
    for k in range(8):
        h = h + feats[k] * w1r_ref[k:k + 1, :]
    h = jax.nn.relu(h + b1_ref[...])
    m = jnp.dot(h[:, 0:F], ew2_ref[...], preferred_element_type=jnp.float32)
    m = m + eb2_ref[...]
    wl = jnp.dot(h[:, F:2 * F], ww2_ref[...],
                 preferred_element_type=jnp.float32)
    w = jax.nn.sigmoid(wl[:, 0:1] + wb2_ref[:, 0:1])
    out_ref[...] = w * m


_EDGE_GRID = (E // BE,)
_EDGE_IN_SPECS = [
    pl.BlockSpec((BE, F), lambda i: (i, 0)),
    pl.BlockSpec((BE, F), lambda i: (i, 0)),
    pl.BlockSpec((F, 2 * F), lambda i: (0, 0)),
    pl.BlockSpec((F, 2 * F), lambda i: (0, 0)),
    pl.BlockSpec((8, 2 * F), lambda i: (0, 0)),
    pl.BlockSpec((1, 2 * F), lambda i: (0, 0)),
    pl.BlockSpec((F, F), lambda i: (0, 0)),
    pl.BlockSpec((1, F), lambda i: (0, 0)),
    pl.BlockSpec((F, F), lambda i: (0, 0)),
    pl.BlockSpec((1, F), lambda i: (0, 0)),
]
_EDGE_OUT_SPEC = pl.BlockSpec((BE, F), lambda i: (i, 0))
_EDGE_OUT_SHAPE = jax.ShapeDtypeStruct((E, F), jnp.float32)

_edge_call = pl.pallas_call(
    _edge_body,
    grid=_EDGE_GRID,
    in_specs=_EDGE_IN_SPECS,
    out_specs=_EDGE_OUT_SPEC,
    out_shape=_EDGE_OUT_SHAPE,
)


def _world_body(z_ref, pos_ref, gw1a_ref, gw1r_ref, gb1_ref, gw2_ref,
                gb2_ref, out_ref):
    z = z_ref[...]
    posterm = jnp.dot(pos_ref[...], gw1r_ref[...],
                      preferred_element_type=jnp.float32)
    h = jnp.dot(z, gw1a_ref[...], preferred_element_type=jnp.float32)
    for k in range(3):
        h = h + z[:, k:k + 1] * gw1r_ref[k:k + 1, :]
    h = jax.nn.relu(h - posterm + gb1_ref[...])
    mw = jnp.dot(h, gw2_ref[...], preferred_element_type=jnp.float32)
    mw = mw + gb2_ref[...]
    out_ref[...] = jnp.sum(mw, axis=0, keepdims=True)


_world_call = pl.pallas_call(
    _world_body,
    out_shape=jax.ShapeDtypeStruct((1, F), jnp.float32),
)


NBL = 1000


def _node_body(z_ref, p0_ref, p1_ref, mw_ref, nw1a_ref, nw1b_ref, nb1_ref,
               nw2_ref, nb2_ref, out_ref):
    z = z_ref[...]
    magg = p0_ref[...] + p1_ref[...] + mw_ref[...]
    h = jnp.dot(z, nw1a_ref[...], preferred_element_type=jnp.float32)
    h = h + jnp.dot(magg, nw1b_ref[...], preferred_element_type=jnp.float32)
    h = jax.nn.relu(h + nb1_ref[...])
    out = jnp.dot(h, nw2_ref[...], preferred_element_type=jnp.float32)
    out_ref[...] = out + nb2_ref[...]


_NODE_GRID = (N // NBL,)
_NODE_IN_SPECS = [
    pl.BlockSpec((NBL, F), lambda i: (i, 0)),
    pl.BlockSpec((NBL, F), lambda i: (i, 0)),
    pl.BlockSpec((NBL, F), lambda i: (i, 0)),
    pl.BlockSpec((1, F), lambda i: (0, 0)),
    pl.BlockSpec((F, F), lambda i: (0, 0)),
    pl.BlockSpec((F, F), lambda i: (0, 0)),
    pl.BlockSpec((1, F), lambda i: (0, 0)),
    pl.BlockSpec((F, F), lambda i: (0, 0)),
    pl.BlockSpec((1, F), lambda i: (0, 0)),
]
_NODE_OUT_SPEC = pl.BlockSpec((NBL, F), lambda i: (i, 0))
_NODE_OUT_SHAPE = jax.ShapeDtypeStruct((N, F), jnp.float32)

_node_call = pl.pallas_call(
    _node_body,
    grid=_NODE_GRID,
    in_specs=_NODE_IN_SPECS,
    out_specs=_NODE_OUT_SPEC,
    out_shape=_NODE_OUT_SHAPE,
)


def kernel(z_h, edge_index_h_h, pos_world,
           eW1, eb1, eW2, eb2,
           wW1, wb1, wW2, wb2,
           nW1, nb1, nW2, nb2,
           gW1, gb1, gW2, gb2):
    src3 = edge_index_h_h[0].reshape(NW, NGB, GB)
    tgt3 = edge_index_h_h[1].reshape(NW, NGB, GB)

    gather_k, scatter_k = _sc_kernels()
    gs, gt = gather_k(z_h, src3, tgt3)

    w1a = jnp.concatenate([eW1[0:F], wW1[0:F]], axis=1)
    w1b = jnp.concatenate([eW1[F:2 * F], wW1[F:2 * F]], axis=1)
    w1r = jnp.concatenate([eW1[2 * F:], wW1[2 * F:]], axis=1)
    b1 = jnp.concatenate([eb1, wb1]).reshape(1, 2 * F)
    eb2r = eb2.reshape(1, F)
    wb2r = jnp.broadcast_to(wb2.reshape(1, 1), (1, F))
    ww2p = jnp.zeros((F, F), jnp.float32).at[:, 0:1].set(wW2)
    val = _edge_call(gs, gt, w1a, w1b, w1r, b1, eW2, eb2r, ww2p, wb2r)

    zeros = jnp.zeros((N, F), jnp.float32)
    parts = scatter_k(val, tgt3, zeros)

    posp = jnp.zeros((1, F), jnp.float32).at[0, 0:3].set(pos_world[0])
    gw1rp = jnp.zeros((F, F), jnp.float32).at[0:3].set(gW1[F:F + 3])
    mw = _world_call(z_h, posp, gW1[0:F], gw1rp, gb1.reshape(1, F), gW2,
                     gb2.reshape(1, F))

    return _node_call(z_h, parts[0], parts[1], mw, nW1[0:F], nW1[F:2 * F],
                      nb1.reshape(1, F), nW2, nb2.reshape(1, F))


# speedup vs baseline: 3.6939x; 1.3247x over previous
import functools

import jax
import jax.numpy as jnp
from jax import lax
from jax.experimental import pallas as pl
from jax.experimental.pallas import tpu as pltpu, tpu_sc as plsc

N = 10000
E = 320000
F = 128

NW = 32
EPW = E // NW
GB = 80
NGB = EPW // GB
RPS = 1000

def _gather_body(z_hbm, src_hbm, tgt_hbm, out_s, out_t, sidx, tidx, bufs,
                 buft, sem_s, sem_t):
    wid = lax.axis_index("s") * 2 + lax.axis_index("c")
    base = wid * EPW
    pltpu.sync_copy(src_hbm.at[wid], sidx)
    pltpu.sync_copy(tgt_hbm.at[wid], tidx)

    def body(j, carry):
        cs = pltpu.async_copy(z_hbm.at[sidx.at[j]], bufs, sem_s)
        ct = pltpu.async_copy(z_hbm.at[tidx.at[j]], buft, sem_t)
        cs.wait()
        pltpu.sync_copy(bufs, out_s.at[pl.ds(base + j * GB, GB)])
        ct.wait()
        pltpu.sync_copy(buft, out_t.at[pl.ds(base + j * GB, GB)])
        return carry

    lax.fori_loop(0, NGB, body, 0)


def _scatter_body(val_hbm, tgt_hbm, zero_hbm, out_hbm, tidx, vbuf, acc):
    c = lax.axis_index("c")
    s = lax.axis_index("s")
    wid = s * 2 + c
    base = wid * EPW

    @pl.when(s < N // RPS)
    def _():
        pltpu.sync_copy(zero_hbm.at[pl.ds(s * RPS, RPS)],
                        acc.at[pl.ds(s * RPS, RPS)])

    plsc.subcore_barrier()
    pltpu.sync_copy(tgt_hbm.at[wid], tidx)

    def body(j, carry):
        pltpu.sync_copy(val_hbm.at[pl.ds(base + j * GB, GB)], vbuf)
        pltpu.sync_copy(vbuf, acc.at[tidx.at[j]], add=True)
        return carry

    lax.fori_loop(0, NGB, body, 0)
    plsc.subcore_barrier()

    @pl.when(s < N // RPS)
    def _():
        pltpu.sync_copy(acc.at[pl.ds(s * RPS, RPS)],
                        out_hbm.at[c, pl.ds(s * RPS, RPS)])


@functools.cache
def _sc_kernels():
    mesh = plsc.VectorSubcoreMesh(core_axis_name="c", subcore_axis_name="s")
    gather = pl.kernel(
        _gather_body,
        out_type=(
            jax.ShapeDtypeStruct((E, F), jnp.float32),
            jax.ShapeDtypeStruct((E, F), jnp.float32),
        ),
        mesh=mesh,
        scratch_types=[
            pltpu.VMEM((NGB, GB), jnp.int32),
            pltpu.VMEM((NGB, GB), jnp.int32),
            pltpu.VMEM((GB, F), jnp.float32),
            pltpu.VMEM((GB, F), jnp.float32),
            pltpu.SemaphoreType.DMA,
            pltpu.SemaphoreType.DMA,
        ],
    )
    scatter = pl.kernel(
        _scatter_body,
        out_type=jax.ShapeDtypeStruct((2, N, F), jnp.float32),
        mesh=mesh,
        scratch_types=[
            pltpu.VMEM((NGB, GB), jnp.int32),
            pltpu.VMEM((GB, F), jnp.float32),
            pltpu.VMEM_SHARED((N, F), jnp.float32),
        ],
    )
    return gather, scatter


BE = 2000


def _roll(x, k):
    return jnp.concatenate([x[:, k:], x[:, :k]], axis=1)


def _edge_body(gs_ref, gt_ref, w1a_ref, w1b_ref, mc_ref, mqb_ref, w1r7_ref,
               b1_ref, ew2_ref, eb2_ref, ww2t_ref, wb2_ref, out_ref):
    zs = gs_ref[...]
    zt = gt_ref[...]
    lane = lax.broadcasted_iota(jnp.int32, (BE, F), 1)
    dz = zs - zt
    a1, a2 = _roll(zs, 1), _roll(zs, 2)
    am1, am2 = _roll(zs, -1), _roll(zs, -2)
    b1r, b2r = _roll(zt, 1), _roll(zt, 2)
    bm1, bm2 = _roll(zt, -1), _roll(zt, -2)
    candA = a1 * b2r - a2 * b1r
    candB = a1 * bm1 - am1 * b1r
    candC = am2 * bm1 - am1 * bm2
    pre = jnp.where(lane < 3, dz,
                    jnp.where(lane == 3, candA,
                              jnp.where(lane == 4, candB, candC)))
    sq = pre * pre
    zsb = zs.astype(jnp.bfloat16)
    ztb = zt.astype(jnp.bfloat16)
    h = jnp.dot(zsb, w1a_ref[...], preferred_element_type=jnp.float32)
    h = h + jnp.dot(ztb, w1b_ref[...], preferred_element_type=jnp.float32)
    h = h + jnp.dot(pre.astype(jnp.bfloat16), mc_ref[...],
                    preferred_element_type=jnp.float32)
    sqout = jnp.dot(sq.astype(jnp.bfloat16), mqb_ref[...],
                    preferred_element_type=jnp.float32)
    h = h + sqout[:, 0:2 * F]
    cn = jnp.sqrt(sqout[:, 2 * F:4 * F])
    h = h + cn * w1r7_ref[...]
    h = jax.nn.relu(h + b1_ref[...]).astype(jnp.bfloat16)
    m = jnp.dot(h[:, 0:F], ew2_ref[...], preferred_element_type=jnp.float32)
    m = m + eb2_ref[...]
    wl = jnp.dot(h[:, F:2 * F], ww2t_ref[...],
                 preferred_element_type=jnp.float32)
    w = jax.nn.sigmoid(wl + wb2_ref[...])
    out_ref[...] = w * m


_EDGE_GRID = (E // BE,)
_EDGE_IN_SPECS = [
    pl.BlockSpec((BE, F), lambda i: (i, 0)),
    pl.BlockSpec((BE, F), lambda i: (i, 0)),
    pl.BlockSpec((F, 2 * F), lambda i: (0, 0)),
    pl.BlockSpec((F, 2 * F), lambda i: (0, 0)),
    pl.BlockSpec((F, 2 * F), lambda i: (0, 0)),
    pl.BlockSpec((F, 4 * F), lambda i: (0, 0)),
    pl.BlockSpec((1, 2 * F), lambda i: (0, 0)),
    pl.BlockSpec((1, 2 * F), lambda i: (0, 0)),
    pl.BlockSpec((F, F), lambda i: (0, 0)),
    pl.BlockSpec((1, F), lambda i: (0, 0)),
    pl.BlockSpec((F, F), lambda i: (0, 0)),
    pl.BlockSpec((1, F), lambda i: (0, 0)),
]
_EDGE_OUT_SPEC = pl.BlockSpec((BE, F), lambda i: (i, 0))
_EDGE_OUT_SHAPE = jax.ShapeDtypeStruct((E, F), jnp.float32)

_edge_call = pl.pallas_call(
    _edge_body,
    grid=_EDGE_GRID,
    in_specs=_EDGE_IN_SPECS,
    out_specs=_EDGE_OUT_SPEC,
    out_shape=_EDGE_OUT_SHAPE,
)


def _world_body(z_ref, pos_ref, gw1a_ref, gw1r_ref, gb1_ref, gw2_ref,
                gb2_ref, out_ref):
    z = z_ref[...]
    posterm = jnp.dot(pos_ref[...], gw1r_ref[...],
                      preferred_element_type=jnp.float32)
    h = jnp.dot(z, gw1a_ref[...], preferred_element_type=jnp.float32)
    for k in range(3):
        h = h + z[:, k:k + 1] * gw1r_ref[k:k + 1, :]
    h = jax.nn.relu(h - posterm + gb1_ref[...])
    mw = jnp.dot(h, gw2_ref[...], preferred_element_type=jnp.float32)
    mw = mw + gb2_ref[...]
    out_ref[...] = jnp.sum(mw, axis=0, keepdims=True)


_world_call = pl.pallas_call(
    _world_body,
    out_shape=jax.ShapeDtypeStruct((1, F), jnp.float32),
)


NBL = 1000


def _node_body(z_ref, p0_ref, p1_ref, mw_ref, nw1a_ref, nw1b_ref, nb1_ref,
               nw2_ref, nb2_ref, out_ref):
    z = z_ref[...]
    magg = p0_ref[...] + p1_ref[...] + mw_ref[...]
    h = jnp.dot(z, nw1a_ref[...], preferred_element_type=jnp.float32)
    h = h + jnp.dot(magg, nw1b_ref[...], preferred_element_type=jnp.float32)
    h = jax.nn.relu(h + nb1_ref[...])
    out = jnp.dot(h, nw2_ref[...], preferred_element_type=jnp.float32)
    out_ref[...] = out + nb2_ref[...]


_NODE_GRID = (N // NBL,)
_NODE_IN_SPECS = [
    pl.BlockSpec((NBL, F), lambda i: (i, 0)),
    pl.BlockSpec((NBL, F), lambda i: (i, 0)),
    pl.BlockSpec((NBL, F), lambda i: (i, 0)),
    pl.BlockSpec((1, F), lambda i: (0, 0)),
    pl.BlockSpec((F, F), lambda i: (0, 0)),
    pl.BlockSpec((F, F), lambda i: (0, 0)),
    pl.BlockSpec((1, F), lambda i: (0, 0)),
    pl.BlockSpec((F, F), lambda i: (0, 0)),
    pl.BlockSpec((1, F), lambda i: (0, 0)),
]
_NODE_OUT_SPEC = pl.BlockSpec((NBL, F), lambda i: (i, 0))
_NODE_OUT_SHAPE = jax.ShapeDtypeStruct((N, F), jnp.float32)

_node_call = pl.pallas_call(
    _node_body,
    grid=_NODE_GRID,
    in_specs=_NODE_IN_SPECS,
    out_specs=_NODE_OUT_SPEC,
    out_shape=_NODE_OUT_SHAPE,
)


def kernel(z_h, edge_index_h_h, pos_world,
           eW1, eb1, eW2, eb2,
           wW1, wb1, wW2, wb2,
           nW1, nb1, nW2, nb2,
           gW1, gb1, gW2, gb2):
    src3 = edge_index_h_h[0].reshape(NW, NGB, GB)
    tgt3 = edge_index_h_h[1].reshape(NW, NGB, GB)

    gather_k, scatter_k = _sc_kernels()
    gs, gt = gather_k(z_h, src3, tgt3)

    w1r = jnp.concatenate([eW1[2 * F:], wW1[2 * F:]], axis=1)
    z3w = jnp.zeros((F, 2 * F), jnp.float32).at[0:3].set(w1r[0:3])
    w1a = (jnp.concatenate([eW1[0:F], wW1[0:F]], axis=1)
           + z3w).astype(jnp.bfloat16)
    w1b = (jnp.concatenate([eW1[F:2 * F], wW1[F:2 * F]], axis=1)
           - z3w).astype(jnp.bfloat16)
    mc = jnp.zeros((F, 2 * F), jnp.float32).at[3:6].set(
        w1r[4:7]).astype(jnp.bfloat16)
    mqb = jnp.zeros((F, 4 * F), jnp.float32)
    mqb = mqb.at[0:3, 0:2 * F].set(jnp.broadcast_to(w1r[3:4], (3, 2 * F)))
    mqb = mqb.at[3:6, 2 * F:4 * F].set(1.0).astype(jnp.bfloat16)
    w1r7 = w1r[7:8]
    b1 = jnp.concatenate([eb1, wb1]).reshape(1, 2 * F)
    eb2r = eb2.reshape(1, F)
    wb2r = jnp.broadcast_to(wb2.reshape(1, 1), (1, F))
    ww2t = jnp.broadcast_to(wW2, (F, F)).astype(jnp.bfloat16)
    val = _edge_call(gs, gt, w1a, w1b, mc, mqb, w1r7, b1,
                     eW2.astype(jnp.bfloat16), eb2r, ww2t, wb2r)

    zeros = jnp.zeros((N, F), jnp.float32)
    parts = scatter_k(val, tgt3, zeros)

    posp = jnp.zeros((1, F), jnp.float32).at[0, 0:3].set(pos_world[0])
    gw1rp = jnp.zeros((F, F), jnp.float32).at[0:3].set(gW1[F:F + 3])
    mw = _world_call(z_h, posp, gW1[0:F], gw1rp, gb1.reshape(1, F), gW2,
                     gb2.reshape(1, F))

    return _node_call(z_h, parts[0], parts[1], mw, nW1[0:F], nW1[F:2 * F],
                      nb1.reshape(1, F), nW2, nb2.reshape(1, F))


# speedup vs baseline: 4.7034x; 1.2733x over previous
import functools

import jax
import jax.numpy as jnp
from jax import lax
from jax.experimental import pallas as pl
from jax.experimental.pallas import tpu as pltpu, tpu_sc as plsc

N = 10000
E = 320000
F = 128

NW = 32
EPW = E // NW
GB = 80
NGB = EPW // GB
RPS = 1000

def _gather_body(z_hbm, src_hbm, tgt_hbm, out_s, out_t, sidx, tidx,
                 bs0, bs1, bt0, bt1, ss0, ss1, st0, st1):
    wid = lax.axis_index("s") * 2 + lax.axis_index("c")
    base = wid * EPW
    pltpu.sync_copy(src_hbm.at[wid], sidx)
    pltpu.sync_copy(tgt_hbm.at[wid], tidx)

    def drain(buf, sem):
        pltpu.make_async_copy(z_hbm.at[pl.ds(0, GB)], buf, sem).wait()

    def emit(j, buf, side_out):
        drain(*buf)
        pltpu.sync_copy(buf[0], side_out.at[pl.ds(base + j * GB, GB)])

    pltpu.async_copy(z_hbm.at[sidx.at[0]], bs0, ss0)
    pltpu.async_copy(z_hbm.at[tidx.at[0]], bt0, st0)

    def body(i, carry):
        j0 = 2 * i
        pltpu.async_copy(z_hbm.at[sidx.at[j0 + 1]], bs1, ss1)
        pltpu.async_copy(z_hbm.at[tidx.at[j0 + 1]], bt1, st1)
        emit(j0, (bs0, ss0), out_s)
        emit(j0, (bt0, st0), out_t)
        pltpu.async_copy(z_hbm.at[sidx.at[j0 + 2]], bs0, ss0)
        pltpu.async_copy(z_hbm.at[tidx.at[j0 + 2]], bt0, st0)
        emit(j0 + 1, (bs1, ss1), out_s)
        emit(j0 + 1, (bt1, st1), out_t)
        return carry

    lax.fori_loop(0, (NGB - 1) // 2, body, 0)
    emit(NGB - 1, (bs0, ss0), out_s)
    emit(NGB - 1, (bt0, st0), out_t)


def _scatter_body(val_hbm, tgt_hbm, zero_hbm, out_hbm, tidx, vb0, vb1, acc,
                  rs0, rs1):
    c = lax.axis_index("c")
    s = lax.axis_index("s")
    wid = s * 2 + c
    base = wid * EPW

    @pl.when(s < N // RPS)
    def _():
        pltpu.sync_copy(zero_hbm.at[pl.ds(s * RPS, RPS)],
                        acc.at[pl.ds(s * RPS, RPS)])

    plsc.subcore_barrier()
    pltpu.sync_copy(tgt_hbm.at[wid], tidx)

    def scat(j, buf, sem):
        pltpu.make_async_copy(val_hbm.at[pl.ds(0, GB)], buf, sem).wait()
        pltpu.sync_copy(buf, acc.at[tidx.at[j]], add=True)

    pltpu.async_copy(val_hbm.at[pl.ds(base, GB)], vb0, rs0)

    def body(i, carry):
        j0 = 2 * i
        pltpu.async_copy(val_hbm.at[pl.ds(base + (j0 + 1) * GB, GB)],
                         vb1, rs1)
        scat(j0, vb0, rs0)
        pltpu.async_copy(val_hbm.at[pl.ds(base + (j0 + 2) * GB, GB)],
                         vb0, rs0)
        scat(j0 + 1, vb1, rs1)
        return carry

    lax.fori_loop(0, (NGB - 1) // 2, body, 0)
    scat(NGB - 1, vb0, rs0)
    plsc.subcore_barrier()

    @pl.when(s < N // RPS)
    def _():
        pltpu.sync_copy(acc.at[pl.ds(s * RPS, RPS)],
                        out_hbm.at[c, pl.ds(s * RPS, RPS)])


@functools.cache
def _sc_kernels():
    mesh = plsc.VectorSubcoreMesh(core_axis_name="c", subcore_axis_name="s")
    gather = pl.kernel(
        _gather_body,
        out_type=(
            jax.ShapeDtypeStruct((E, F), jnp.float32),
            jax.ShapeDtypeStruct((E, F), jnp.float32),
        ),
        mesh=mesh,
        scratch_types=[
            pltpu.VMEM((NGB, GB), jnp.int32),
            pltpu.VMEM((NGB, GB), jnp.int32),
            pltpu.VMEM((GB, F), jnp.float32),
            pltpu.VMEM((GB, F), jnp.float32),
            pltpu.VMEM((GB, F), jnp.float32),
            pltpu.VMEM((GB, F), jnp.float32),
            pltpu.SemaphoreType.DMA,
            pltpu.SemaphoreType.DMA,
            pltpu.SemaphoreType.DMA,
            pltpu.SemaphoreType.DMA,
        ],
    )
    scatter = pl.kernel(
        _scatter_body,
        out_type=jax.ShapeDtypeStruct((2, N, F), jnp.float32),
        mesh=mesh,
        scratch_types=[
            pltpu.VMEM((NGB, GB), jnp.int32),
            pltpu.VMEM((GB, F), jnp.float32),
            pltpu.VMEM((GB, F), jnp.float32),
            pltpu.VMEM_SHARED((N, F), jnp.float32),
            pltpu.SemaphoreType.DMA,
            pltpu.SemaphoreType.DMA,
        ],
    )
    return gather, scatter


BE = 2000


def _roll(x, k):
    return jnp.concatenate([x[:, k:], x[:, :k]], axis=1)


def _edge_body(gs_ref, gt_ref, wab_ref, wu_ref, w1r7_ref, b1_ref, w2bd_ref,
               b2_ref, out_ref):
    zs = gs_ref[...]
    zt = gt_ref[...]
    lane = lax.broadcasted_iota(jnp.int32, (BE, F), 1)
    dz = zs - zt
    a1, a2 = _roll(zs, 1), _roll(zs, 2)
    am1, am2 = _roll(zs, -1), _roll(zs, -2)
    b1r, b2r = _roll(zt, 1), _roll(zt, 2)
    bm1, bm2 = _roll(zt, -1), _roll(zt, -2)
    candA = a1 * b2r - a2 * b1r
    candB = a1 * bm1 - am1 * b1r
    candC = am2 * bm1 - am1 * bm2
    pre = jnp.where(lane < 3, dz,
                    jnp.where(lane == 3, candA,
                              jnp.where(lane == 4, candB, candC)))
    sq = pre * pre
    x = jnp.concatenate([zs, zt], axis=1).astype(jnp.bfloat16)
    u = jnp.concatenate([pre, sq], axis=1).astype(jnp.bfloat16)
    h = jnp.dot(x, wab_ref[...], preferred_element_type=jnp.float32)
    uo = jnp.dot(u, wu_ref[...], preferred_element_type=jnp.float32)
    cn = jnp.sqrt(uo[:, 2 * F:3 * F])
    cnb = jnp.concatenate([cn, cn], axis=1)
    h = h + uo[:, 0:2 * F] + cnb * w1r7_ref[...] + b1_ref[...]
    h = jax.nn.relu(h).astype(jnp.bfloat16)
    mo = jnp.dot(h, w2bd_ref[...], preferred_element_type=jnp.float32)
    mo = mo + b2_ref[...]
    w = jax.nn.sigmoid(mo[:, F:2 * F])
    out_ref[...] = w * mo[:, 0:F]


_EDGE_GRID = (E // BE,)
_EDGE_IN_SPECS = [
    pl.BlockSpec((BE, F), lambda i: (i, 0)),
    pl.BlockSpec((BE, F), lambda i: (i, 0)),
    pl.BlockSpec((2 * F, 2 * F), lambda i: (0, 0)),
    pl.BlockSpec((2 * F, 3 * F), lambda i: (0, 0)),
    pl.BlockSpec((1, 2 * F), lambda i: (0, 0)),
    pl.BlockSpec((1, 2 * F), lambda i: (0, 0)),
    pl.BlockSpec((2 * F, 2 * F), lambda i: (0, 0)),
    pl.BlockSpec((1, 2 * F), lambda i: (0, 0)),
]
_EDGE_OUT_SPEC = pl.BlockSpec((BE, F), lambda i: (i, 0))
_EDGE_OUT_SHAPE = jax.ShapeDtypeStruct((E, F), jnp.float32)

_edge_call = pl.pallas_call(
    _edge_body,
    grid=_EDGE_GRID,
    in_specs=_EDGE_IN_SPECS,
    out_specs=_EDGE_OUT_SPEC,
    out_shape=_EDGE_OUT_SHAPE,
)


def _world_body(z_ref, pos_ref, gw1a_ref, gw1r_ref, gb1_ref, gw2_ref,
                gb2_ref, out_ref):
    z = z_ref[...]
    posterm = jnp.dot(pos_ref[...], gw1r_ref[...],
                      preferred_element_type=jnp.float32)
    h = jnp.dot(z, gw1a_ref[...], preferred_element_type=jnp.float32)
    for k in range(3):
        h = h + z[:, k:k + 1] * gw1r_ref[k:k + 1, :]
    h = jax.nn.relu(h - posterm + gb1_ref[...])
    mw = jnp.dot(h, gw2_ref[...], preferred_element_type=jnp.float32)
    mw = mw + gb2_ref[...]
    out_ref[...] = jnp.sum(mw, axis=0, keepdims=True)


_world_call = pl.pallas_call(
    _world_body,
    out_shape=jax.ShapeDtypeStruct((1, F), jnp.float32),
)


NBL = 1000


def _node_body(z_ref, p0_ref, p1_ref, mw_ref, nw1a_ref, nw1b_ref, nb1_ref,
               nw2_ref, nb2_ref, out_ref):
    z = z_ref[...]
    magg = p0_ref[...] + p1_ref[...] + mw_ref[...]
    h = jnp.dot(z, nw1a_ref[...], preferred_element_type=jnp.float32)
    h = h + jnp.dot(magg, nw1b_ref[...], preferred_element_type=jnp.float32)
    h = jax.nn.relu(h + nb1_ref[...])
    out = jnp.dot(h, nw2_ref[...], preferred_element_type=jnp.float32)
    out_ref[...] = out + nb2_ref[...]


_NODE_GRID = (N // NBL,)
_NODE_IN_SPECS = [
    pl.BlockSpec((NBL, F), lambda i: (i, 0)),
    pl.BlockSpec((NBL, F), lambda i: (i, 0)),
    pl.BlockSpec((NBL, F), lambda i: (i, 0)),
    pl.BlockSpec((1, F), lambda i: (0, 0)),
    pl.BlockSpec((F, F), lambda i: (0, 0)),
    pl.BlockSpec((F, F), lambda i: (0, 0)),
    pl.BlockSpec((1, F), lambda i: (0, 0)),
    pl.BlockSpec((F, F), lambda i: (0, 0)),
    pl.BlockSpec((1, F), lambda i: (0, 0)),
]
_NODE_OUT_SPEC = pl.BlockSpec((NBL, F), lambda i: (i, 0))
_NODE_OUT_SHAPE = jax.ShapeDtypeStruct((N, F), jnp.float32)

_node_call = pl.pallas_call(
    _node_body,
    grid=_NODE_GRID,
    in_specs=_NODE_IN_SPECS,
    out_specs=_NODE_OUT_SPEC,
    out_shape=_NODE_OUT_SHAPE,
)


def kernel(z_h, edge_index_h_h, pos_world,
           eW1, eb1, eW2, eb2,
           wW1, wb1, wW2, wb2,
           nW1, nb1, nW2, nb2,
           gW1, gb1, gW2, gb2):
    src3 = edge_index_h_h[0].reshape(NW, NGB, GB)
    tgt3 = edge_index_h_h[1].reshape(NW, NGB, GB)

    gather_k, scatter_k = _sc_kernels()
    gs, gt = gather_k(z_h, src3, tgt3)

    w1r = jnp.concatenate([eW1[2 * F:], wW1[2 * F:]], axis=1)
    z3w = jnp.zeros((F, 2 * F), jnp.float32).at[0:3].set(w1r[0:3])
    w1a = jnp.concatenate([eW1[0:F], wW1[0:F]], axis=1) + z3w
    w1b = jnp.concatenate([eW1[F:2 * F], wW1[F:2 * F]], axis=1) - z3w
    wab = jnp.concatenate([w1a, w1b], axis=0).astype(jnp.bfloat16)
    wu = jnp.zeros((2 * F, 3 * F), jnp.float32)
    wu = wu.at[3:6, 0:2 * F].set(w1r[4:7])
    wu = wu.at[F:F + 3, 0:2 * F].set(
        jnp.broadcast_to(w1r[3:4], (3, 2 * F)))
    wu = wu.at[F + 3:F + 6, 2 * F:3 * F].set(1.0)
    wu = wu.astype(jnp.bfloat16)
    w1r7 = w1r[7:8]
    b1 = jnp.concatenate([eb1, wb1]).reshape(1, 2 * F)
    w2bd = jnp.zeros((2 * F, 2 * F), jnp.float32)
    w2bd = w2bd.at[0:F, 0:F].set(eW2)
    w2bd = w2bd.at[F:2 * F, F:2 * F].set(jnp.broadcast_to(wW2, (F, F)))
    w2bd = w2bd.astype(jnp.bfloat16)
    b2 = jnp.concatenate([eb2, jnp.broadcast_to(wb2, (F,))]).reshape(1, 2 * F)
    val = _edge_call(gs, gt, wab, wu, w1r7, b1, w2bd, b2)

    zeros = jnp.zeros((N, F), jnp.float32)
    parts = scatter_k(val, tgt3, zeros)

    posp = jnp.zeros((1, F), jnp.float32).at[0, 0:3].set(pos_world[0])
    gw1rp = jnp.zeros((F, F), jnp.float32).at[0:3].set(gW1[F:F + 3])
    mw = _world_call(z_h, posp, gW1[0:F], gw1rp, gb1.reshape(1, F), gW2,
                     gb2.reshape(1, F))

    return _node_call(z_h, parts[0], parts[1], mw, nW1[0:F], nW1[F:2 * F],
                      nb1.reshape(1, F), nW2, nb2.reshape(1, F))
